# Initial kernel scaffold; baseline (speedup 1.0000x reference)
import jax
import jax.numpy as jnp
from jax.experimental import pallas as pl


def kernel(x, edge_index, W1, b1, W2, b2):
    raise NotImplementedError("write your pallas kernel here")


# speedup vs baseline: 12.7718x; 12.7718x over previous
import jax
import jax.numpy as jnp
from jax import lax
from jax.experimental import pallas as pl
from jax.experimental.pallas import tpu as pltpu
from jax.experimental.pallas import tpu_sc as plsc

N = 10000
E = 320000
IN_CH = 128
D1 = 128
D2 = 64

NC, NS, LANES = 2, 16, 16
CHUNK = 128
NCHUNK = 79
EPT = NCHUNK * CHUNK
E_PAD = EPT * NC * NS
ACC_ROWS = 10240
RPS = ACC_ROWS // NS

_mesh = plsc.VectorSubcoreMesh(core_axis_name="c", subcore_axis_name="s")


def _deg_body(dst_hbm, out_hbm, ibuf, hist):
    wid = lax.axis_index("s") * NC + lax.axis_index("c")

    @pl.loop(0, ACC_ROWS // LANES)
    def _(i):
        hist[pl.ds(i * LANES, LANES)] = jnp.zeros((LANES,), jnp.float32)

    ones = jnp.ones((LANES,), jnp.float32)
    base0 = wid * EPT

    @pl.loop(0, NCHUNK)
    def _(i):
        pltpu.sync_copy(dst_hbm.at[pl.ds(base0 + i * CHUNK, CHUNK)], ibuf.at[0])

        @pl.loop(0, CHUNK // LANES)
        def _(j):
            idx = ibuf[0, pl.ds(j * LANES, LANES)]
            plsc.addupdate_scatter(hist, [idx], ones)

    pltpu.sync_copy(hist, out_hbm.at[wid])


def _compute_deg_parts(dst_pad):
    return pl.kernel(
        _deg_body,
        out_type=jax.ShapeDtypeStruct((NC * NS, ACC_ROWS), jnp.float32),
        mesh=_mesh,
        scratch_types=[
            pltpu.VMEM((1, CHUNK), jnp.int32),
            pltpu.VMEM((ACC_ROWS,), jnp.float32),
        ],
        compiler_params=pltpu.CompilerParams(needs_layout_passes=False),
    )(dst_pad)


def _make_agg_body(d):
    def body(h_hbm, src_hbm, dst_hbm, out_hbm, sidx, didx, rows, acc, gsem):
        cid = lax.axis_index("c")
        sid = lax.axis_index("s")

        @pl.loop(0, CHUNK)
        def _(r):
            @pl.loop(0, d // LANES)
            def _(q):
                rows[r, pl.ds(q * LANES, LANES)] = jnp.zeros((LANES,), jnp.float32)

        @pl.loop(0, RPS // CHUNK)
        def _(k):
            pltpu.sync_copy(rows, acc.at[pl.ds(sid * RPS + k * CHUNK, CHUNK)])

        plsc.subcore_barrier()

        base0 = cid * (E_PAD // NC) + sid * EPT

        @pl.loop(0, NCHUNK)
        def _(i):
            base = base0 + i * CHUNK
            pltpu.sync_copy(src_hbm.at[pl.ds(base, CHUNK)], sidx.at[0])
            pltpu.sync_copy(dst_hbm.at[pl.ds(base, CHUNK)], didx.at[0])
            pltpu.async_copy(h_hbm.at[sidx.at[0]], rows, gsem).wait()
            pltpu.sync_copy(rows, acc.at[didx.at[0]], add=True)

        plsc.subcore_barrier()
        pltpu.sync_copy(
            acc.at[pl.ds(sid * RPS, RPS)],
            out_hbm.at[cid].at[pl.ds(sid * RPS, RPS)],
        )

    return body


def _sc_aggregate(h_pad, src_pad, dst_pad, d):
    return pl.kernel(
        _make_agg_body(d),
        out_type=jax.ShapeDtypeStruct((NC, ACC_ROWS, d), jnp.float32),
        mesh=_mesh,
        scratch_types=[
            pltpu.VMEM((1, CHUNK), jnp.int32),
            pltpu.VMEM((1, CHUNK), jnp.int32),
            pltpu.VMEM((CHUNK, d), jnp.float32),
            pltpu.VMEM_SHARED((ACC_ROWS, d), jnp.float32),
            pltpu.SemaphoreType.DMA,
        ],
        compiler_params=pltpu.CompilerParams(use_tc_tiling_on_sc=(d == 128)),
    )(h_pad, src_pad, dst_pad)


def _mm_body(x_ref, w_ref, o_ref):
    o_ref[...] = jnp.dot(x_ref[...], w_ref[...], preferred_element_type=jnp.float32)


def _dinv_body(parts_ref, o_ref):
    deg = jnp.sum(parts_ref[...], axis=0, keepdims=True) + 1.0
    o_ref[...] = lax.rsqrt(deg)


def _scale_body(h_ref, dinv_ref, o_ref):
    o_ref[...] = h_ref[...] * dinv_ref[...]


def _comb1_body(p_ref, hs_ref, dinv_ref, b1_ref, w2_ref, o_ref):
    agg = p_ref[0] + p_ref[1] + hs_ref[...]
    out1 = jnp.maximum(agg * dinv_ref[...] + b1_ref[...], 0.0)
    h2 = jnp.dot(out1, w2_ref[...], preferred_element_type=jnp.float32)
    mask = lax.broadcasted_iota(jnp.int32, (ACC_ROWS, 1), 0) < N
    o_ref[...] = jnp.where(mask, h2 * dinv_ref[...], 0.0)


def _comb2_body(p_ref, hs_ref, dinv_ref, b2_ref, o_ref):
    agg = p_ref[0] + p_ref[1] + hs_ref[...]
    o_ref[...] = agg * dinv_ref[...] + b2_ref[...]


def _sds(shape):
    return jax.ShapeDtypeStruct(shape, jnp.float32)


@jax.jit
def _run(x, edge_index, W1, b1, W2, b2):
    src = edge_index[0].astype(jnp.int32)
    dst = edge_index[1].astype(jnp.int32)
    pad = jnp.full((E_PAD - E,), N, jnp.int32)
    src_pad = jnp.concatenate([src, pad])
    dst_pad = jnp.concatenate([dst, pad])
    x_pad = jnp.pad(x, ((0, ACC_ROWS - N), (0, 0)))
    b1r = b1.reshape(1, D1)
    b2r = b2.reshape(1, D2)

    deg_parts = _compute_deg_parts(dst_pad)
    h1 = pl.pallas_call(_mm_body, out_shape=_sds((ACC_ROWS, D1)))(x_pad, W1)

    dinv_row = pl.pallas_call(_dinv_body, out_shape=_sds((1, ACC_ROWS)))(deg_parts)
    dinv = dinv_row.reshape(ACC_ROWS, 1)

    h1s = pl.pallas_call(_scale_body, out_shape=_sds((ACC_ROWS, D1)))(h1, dinv)
    p1 = _sc_aggregate(h1s, src_pad, dst_pad, D1)
    h2s = pl.pallas_call(_comb1_body, out_shape=_sds((ACC_ROWS, D2)))(
        p1, h1s, dinv, b1r, W2
    )
    p2 = _sc_aggregate(h2s, src_pad, dst_pad, D2)
    out = pl.pallas_call(_comb2_body, out_shape=_sds((ACC_ROWS, D2)))(
        p2, h2s, dinv, b2r
    )
    return out[:N]


def kernel(x, edge_index, W1, b1, W2, b2):
    return _run(x, edge_index, W1, b1, W2, b2)


# speedup vs baseline: 13.6516x; 1.0689x over previous
import jax
import jax.numpy as jnp
from jax import lax
from jax.experimental import pallas as pl
from jax.experimental.pallas import tpu as pltpu
from jax.experimental.pallas import tpu_sc as plsc

N = 10000
E = 320000
IN_CH = 128
D1 = 128
D2 = 64

NC, NS, LANES = 2, 16, 16
CHUNK = 128
NCHUNK = 80
EPT = NCHUNK * CHUNK
E_PAD = EPT * NC * NS
ACC_ROWS = 10240
RPS = ACC_ROWS // NS

_mesh = plsc.VectorSubcoreMesh(core_axis_name="c", subcore_axis_name="s")


def _deg_body(dst_hbm, out_hbm, ibuf, hist):
    wid = lax.axis_index("c") * NS + lax.axis_index("s")

    @pl.loop(0, ACC_ROWS // LANES)
    def _(i):
        hist[pl.ds(i * LANES, LANES)] = jnp.zeros((LANES,), jnp.float32)

    ones = jnp.ones((LANES,), jnp.float32)
    pltpu.sync_copy(dst_hbm.at[wid], ibuf)

    @pl.loop(0, NCHUNK)
    def _(i):
        @pl.loop(0, CHUNK // LANES)
        def _(j):
            idx = ibuf[i, pl.ds(j * LANES, LANES)]
            plsc.addupdate_scatter(hist, [idx], ones)

    pltpu.sync_copy(hist, out_hbm.at[wid])


def _compute_deg_parts(dst3):
    return pl.kernel(
        _deg_body,
        out_type=jax.ShapeDtypeStruct((NC * NS, ACC_ROWS), jnp.float32),
        mesh=_mesh,
        scratch_types=[
            pltpu.VMEM((NCHUNK, CHUNK), jnp.int32),
            pltpu.VMEM((ACC_ROWS,), jnp.float32),
        ],
        compiler_params=pltpu.CompilerParams(needs_layout_passes=False),
    )(dst3)


def _make_agg_body(d):
    def body(h_hbm, src_hbm, dst_hbm, out_hbm, sidx, dbuf_a, dbuf_b, rows_a,
             rows_b, acc, sem_a, sem_b, sem_da, sem_db):
        cid = lax.axis_index("c")
        sid = lax.axis_index("s")
        tid = cid * NS + sid

        @pl.loop(0, CHUNK)
        def _(r):
            @pl.loop(0, d // LANES)
            def _(q):
                rows_a[r, pl.ds(q * LANES, LANES)] = jnp.zeros(
                    (LANES,), jnp.float32)

        @pl.loop(0, RPS // CHUNK)
        def _(k):
            pltpu.sync_copy(rows_a, acc.at[pl.ds(sid * RPS + k * CHUNK, CHUNK)])

        plsc.subcore_barrier()

        pltpu.sync_copy(src_hbm.at[tid], sidx)

        pltpu.async_copy(h_hbm.at[sidx.at[0]], rows_a, sem_a)
        pltpu.async_copy(dst_hbm.at[tid].at[0], dbuf_a.at[0], sem_da)

        @pl.loop(0, NCHUNK // 2)
        def _(k):
            i0 = 2 * k
            pltpu.async_copy(h_hbm.at[sidx.at[i0 + 1]], rows_b, sem_b)
            pltpu.async_copy(dst_hbm.at[tid].at[i0 + 1], dbuf_b.at[0], sem_db)
            pltpu.make_async_copy(h_hbm.at[sidx.at[i0]], rows_a, sem_a).wait()
            pltpu.make_async_copy(
                dst_hbm.at[tid].at[i0], dbuf_a.at[0], sem_da).wait()
            pltpu.sync_copy(rows_a, acc.at[dbuf_a.at[0]], add=True)

            @pl.when(k + 1 < NCHUNK // 2)
            def _():
                pltpu.async_copy(h_hbm.at[sidx.at[i0 + 2]], rows_a, sem_a)
                pltpu.async_copy(
                    dst_hbm.at[tid].at[i0 + 2], dbuf_a.at[0], sem_da)

            pltpu.make_async_copy(
                h_hbm.at[sidx.at[i0 + 1]], rows_b, sem_b).wait()
            pltpu.make_async_copy(
                dst_hbm.at[tid].at[i0 + 1], dbuf_b.at[0], sem_db).wait()
            pltpu.sync_copy(rows_b, acc.at[dbuf_b.at[0]], add=True)

        plsc.subcore_barrier()
        pltpu.sync_copy(
            acc.at[pl.ds(sid * RPS, RPS)],
            out_hbm.at[cid].at[pl.ds(sid * RPS, RPS)],
        )

    return body


def _sc_aggregate(h_pad, src3, dst3, d):
    return pl.kernel(
        _make_agg_body(d),
        out_type=jax.ShapeDtypeStruct((NC, ACC_ROWS, d), jnp.float32),
        mesh=_mesh,
        scratch_types=[
            pltpu.VMEM((NCHUNK, CHUNK), jnp.int32),
            pltpu.VMEM((1, CHUNK), jnp.int32),
            pltpu.VMEM((1, CHUNK), jnp.int32),
            pltpu.VMEM((CHUNK, d), jnp.float32),
            pltpu.VMEM((CHUNK, d), jnp.float32),
            pltpu.VMEM_SHARED((ACC_ROWS, d), jnp.float32),
            pltpu.SemaphoreType.DMA,
            pltpu.SemaphoreType.DMA,
            pltpu.SemaphoreType.DMA,
            pltpu.SemaphoreType.DMA,
        ],
        compiler_params=pltpu.CompilerParams(use_tc_tiling_on_sc=(d == 128)),
    )(h_pad, src3, dst3)


def _mm_body(x_ref, w_ref, o_ref):
    o_ref[...] = jnp.dot(x_ref[...], w_ref[...], preferred_element_type=jnp.float32)


def _dinv_body(parts_ref, o_ref):
    deg = jnp.sum(parts_ref[...], axis=0, keepdims=True) + 1.0
    o_ref[...] = lax.rsqrt(deg)


def _scale_body(h_ref, dinv_ref, o_ref):
    o_ref[...] = h_ref[...] * dinv_ref[...]


def _comb1_body(p_ref, hs_ref, dinv_ref, b1_ref, w2_ref, o_ref):
    agg = p_ref[0] + p_ref[1] + hs_ref[...]
    out1 = jnp.maximum(agg * dinv_ref[...] + b1_ref[...], 0.0)
    h2 = jnp.dot(out1, w2_ref[...], preferred_element_type=jnp.float32)
    mask = lax.broadcasted_iota(jnp.int32, (ACC_ROWS, 1), 0) < N
    o_ref[...] = jnp.where(mask, h2 * dinv_ref[...], 0.0)


def _comb2_body(p_ref, hs_ref, dinv_ref, b2_ref, o_ref):
    agg = p_ref[0] + p_ref[1] + hs_ref[...]
    o_ref[...] = agg * dinv_ref[...] + b2_ref[...]


def _sds(shape):
    return jax.ShapeDtypeStruct(shape, jnp.float32)


@jax.jit
def _run(x, edge_index, W1, b1, W2, b2):
    src = edge_index[0].astype(jnp.int32)
    dst = edge_index[1].astype(jnp.int32)
    pad = jnp.full((E_PAD - E,), N, jnp.int32)
    src3 = jnp.concatenate([src, pad]).reshape(NC * NS, NCHUNK, CHUNK)
    dst3 = jnp.concatenate([dst, pad]).reshape(NC * NS, NCHUNK, CHUNK)
    x_pad = jnp.pad(x, ((0, ACC_ROWS - N), (0, 0)))
    b1r = b1.reshape(1, D1)
    b2r = b2.reshape(1, D2)

    deg_parts = _compute_deg_parts(dst3)
    h1 = pl.pallas_call(_mm_body, out_shape=_sds((ACC_ROWS, D1)))(x_pad, W1)

    dinv_row = pl.pallas_call(_dinv_body, out_shape=_sds((1, ACC_ROWS)))(deg_parts)
    dinv = dinv_row.reshape(ACC_ROWS, 1)

    h1s = pl.pallas_call(_scale_body, out_shape=_sds((ACC_ROWS, D1)))(h1, dinv)
    p1 = _sc_aggregate(h1s, src3, dst3, D1)
    h2s = pl.pallas_call(_comb1_body, out_shape=_sds((ACC_ROWS, D2)))(
        p1, h1s, dinv, b1r, W2
    )
    p2 = _sc_aggregate(h2s, src3, dst3, D2)
    out = pl.pallas_call(_comb2_body, out_shape=_sds((ACC_ROWS, D2)))(
        p2, h2s, dinv, b2r
    )
    return out[:N]


def kernel(x, edge_index, W1, b1, W2, b2):
    return _run(x, edge_index, W1, b1, W2, b2)
